# Initial kernel scaffold; baseline (speedup 1.0000x reference)
import jax
import jax.numpy as jnp
from jax.experimental import pallas as pl


def kernel(seg, instance_map, G, pe2, pe3):
    raise NotImplementedError("write your pallas kernel here")


# speedup vs baseline: 3.1321x; 3.1321x over previous
import functools

import jax
import jax.numpy as jnp
import numpy as np
from jax.experimental import pallas as pl

_B, _H, _W = 8, 256, 256
_K = 16
_EMBED = 256
_NPF = _EMBED // 2


def _or_fold(x, axis):
    n = x.shape[axis]
    while n > 1:
        n //= 2
        if axis == 0:
            x = x[:n] | x[n:]
        else:
            x = x[:, :n] | x[:, n:]
    return x


def _geo_kernel(imap_ref, g_ref, pe2_ref, pe3_ref, out_ref):
    m = imap_ref[0]
    bits = jnp.left_shift(jnp.int32(1), m)

    colbits = _or_fold(bits, 0)
    rowbits = _or_fold(bits, 1)

    ids_row = jax.lax.broadcasted_iota(jnp.int32, (_K, 1), 0) + 1
    ids_col = ids_row

    colk = jnp.bitwise_and(jnp.right_shift(colbits, ids_col), 1)
    xx = jax.lax.broadcasted_iota(jnp.int32, (_K, _W), 1)
    min_x = jnp.min(jnp.where(colk == 1, xx, _W), axis=1, keepdims=True)
    max_x = jnp.max(jnp.where(colk == 1, xx, -1), axis=1, keepdims=True)

    ids_r = jax.lax.broadcasted_iota(jnp.int32, (1, _K), 1) + 1
    rowk = jnp.bitwise_and(jnp.right_shift(rowbits, ids_r), 1)
    yy = jax.lax.broadcasted_iota(jnp.int32, (_H, _K), 0)
    min_y = jnp.min(jnp.where(rowk == 1, yy, _H), axis=0, keepdims=True)
    max_y = jnp.max(jnp.where(rowk == 1, yy, -1), axis=0, keepdims=True)
    min_y = min_y.reshape(_K, 1)
    max_y = max_y.reshape(_K, 1)

    def norm(v, denom):
        return (v.astype(jnp.float32) + 0.5) / denom * 2.0 - 1.0

    c0x = norm(min_x, float(_W))
    c0y = norm(min_y, float(_H))
    c1x = norm(max_x, float(_W))
    c1y = norm(max_y, float(_H))

    g = g_ref[...]
    def b16(v):
        return v.astype(jnp.bfloat16).astype(jnp.float32)

    g0 = b16(g[0:1, :])
    g1 = b16(g[1:2, :])
    c0x, c0y, c1x, c1y = b16(c0x), b16(c0y), b16(c1x), b16(c1y)
    two_pi = jnp.float32(2.0 * np.pi)

    pe0 = two_pi * (c0x * g0 + c0y * g1)
    pe1 = two_pi * (c1x * g0 + c1y * g1)

    emb0 = jnp.concatenate([jnp.sin(pe0), jnp.cos(pe0)], axis=1) + pe2_ref[...]
    emb1 = jnp.concatenate([jnp.sin(pe1), jnp.cos(pe1)], axis=1) + pe3_ref[...]

    out_ref[...] = jnp.concatenate([emb0, emb1], axis=1)


@jax.jit
def _run(instance_map, G, pe2, pe3):
    return pl.pallas_call(
        _geo_kernel,
        grid=(_B,),
        in_specs=[
            pl.BlockSpec((1, _H, _W), lambda b: (b, 0, 0)),
            pl.BlockSpec((2, _NPF), lambda b: (0, 0)),
            pl.BlockSpec((1, _EMBED), lambda b: (0, 0)),
            pl.BlockSpec((1, _EMBED), lambda b: (0, 0)),
        ],
        out_specs=pl.BlockSpec((_K, 2 * _EMBED), lambda b: (b, 0)),
        out_shape=jax.ShapeDtypeStruct((_B * _K, 2 * _EMBED), jnp.float32),
    )(instance_map, G, pe2, pe3)


def kernel(seg, instance_map, G, pe2, pe3):
    del seg
    return _run(instance_map, G, pe2, pe3)
